# Initial kernel scaffold; baseline (speedup 1.0000x reference)
import jax
import jax.numpy as jnp
from jax.experimental import pallas as pl


def kernel(x_center, x_positive, x_negative, table):
    raise NotImplementedError("write your pallas kernel here")


# speedup vs baseline: 2.3579x; 2.3579x over previous
import functools

import jax
import jax.numpy as jnp
from jax import lax
from jax.experimental import pallas as pl
from jax.experimental.pallas import tpu as pltpu
from jax.experimental.pallas import tpu_sc as plsc

DIM = 64
NUM_NEG = 5
NLOG = 1 + NUM_NEG
CHUNK = 128
LANES = 16


@functools.lru_cache(maxsize=None)
def _build_sc_kernel(B: int, NW: int):
    b_per_w = B // NW
    n_chunks = b_per_w // CHUNK
    mesh = plsc.VectorSubcoreMesh(core_axis_name="c", subcore_axis_name="s")

    @functools.partial(
        pl.kernel,
        mesh=mesh,
        compiler_params=pltpu.CompilerParams(
            use_tc_tiling_on_sc=False, needs_layout_passes=False
        ),
        out_type=jax.ShapeDtypeStruct((B * NLOG,), jnp.float32),
        scratch_types=[
            pltpu.VMEM((7, CHUNK), jnp.int32),
            pltpu.VMEM((7, CHUNK, DIM), jnp.float32),
            pltpu.VMEM((CHUNK * NLOG,), jnp.float32),
            pltpu.SemaphoreType.DMA,
        ],
    )
    def sc_kernel(idx_hbm, table_hbm, out_hbm, idx_v, rows_v, out_v, sem):
        wid = lax.axis_index("s") * 2 + lax.axis_index("c")
        lane = lax.iota(jnp.int32, LANES)

        for c in range(n_chunks):
            base = wid * b_per_w + c * CHUNK
            pltpu.sync_copy(idx_hbm.at[wid, c], idx_v)
            cps = [
                pltpu.async_copy(table_hbm.at[idx_v.at[j]], rows_v.at[j], sem)
                for j in range(7)
            ]
            for cp in cps:
                cp.wait()

            def group(g, _):
                bvec = g * LANES + lane
                acc = [jnp.zeros((LANES,), jnp.float32) for _ in range(NLOG)]
                for d in range(DIM):
                    dvec = jnp.full((LANES,), d, jnp.int32)
                    cen = plsc.load_gather(
                        rows_v, [jnp.zeros((LANES,), jnp.int32), bvec, dvec]
                    )
                    for j in range(NLOG):
                        oth = plsc.load_gather(
                            rows_v,
                            [jnp.full((LANES,), 1 + j, jnp.int32), bvec, dvec],
                        )
                        acc[j] = acc[j] + cen * oth
                for j in range(NLOG):
                    prob = 1.0 / (1.0 + jnp.exp(-acc[j]))
                    plsc.store_scatter(out_v, [bvec * NLOG + j], prob)
                return 0

            lax.fori_loop(0, CHUNK // LANES, group, 0)

            pltpu.sync_copy(out_v, out_hbm.at[pl.ds(base * NLOG, CHUNK * NLOG)])

    return sc_kernel


def kernel(x_center, x_positive, x_negative, table):
    B = x_center.shape[0]
    NW = 32
    n_chunks = B // NW // CHUNK
    idx = jnp.concatenate(
        [x_center[None, :], x_positive[None, :], x_negative.T], axis=0
    )
    idx = idx.reshape(7, NW, n_chunks, CHUNK).transpose(1, 2, 0, 3)
    flat = _build_sc_kernel(B, NW)(idx, table)
    return flat.reshape(B, NLOG)
